# baseline (speedup 1.0000x reference)
import functools

import jax
import jax.numpy as jnp
from jax import lax
from jax.experimental import pallas as pl
from jax.experimental.pallas import tpu as pltpu
from jax.experimental.pallas import tpu_sc as plsc

_N = 10000
_D = 128
_H = 256
_C = 2
_NP = 10240
_SLOTS = 16
_RB = 256
_CB = 512
_NEG = -1e30
_FBIG = 2.0**24
_CP = 64


def _topk_body(xr_ref, xc_ref, sqr_ref, sqc_ref, out_ref, runv_ref, runi_ref,
               s_ref, *, k):
    j = pl.program_id(1)

    @pl.when(j == 0)
    def _():
        runv_ref[...] = jnp.full((_RB, _SLOTS), _NEG, jnp.float32)
        runi_ref[...] = jnp.zeros((_RB, _SLOTS), jnp.float32)

    a = xr_ref[...]
    b = xc_ref[...]
    dot = lax.dot_general(a, b, (((1,), (1,)), ((), ())),
                          preferred_element_type=jnp.float32)
    dist = (sqr_ref[...] + sqc_ref[...]) - 2.0 * dot
    colf = (j * _CB + lax.broadcasted_iota(jnp.int32, (_RB, _CB), 1)
            ).astype(jnp.float32)
    s = -dist + jnp.where(colf >= _N, jnp.float32(_NEG), jnp.float32(0.0))

    slotf = lax.broadcasted_iota(jnp.int32, (_RB, _SLOTS), 1
                                 ).astype(jnp.float32)
    kf = jnp.float32(k)

    runv0 = runv_ref[...]
    th = jnp.min(jnp.where(slotf < kf, runv0, -_NEG), axis=1)[:, None]
    cnt = jnp.sum(jnp.where(s > th, 1.0, 0.0), axis=1)
    cmax = jnp.max(jnp.minimum(cnt, kf))

    s_ref[...] = s
    for t in range(k):
        @pl.when(t < cmax)
        def _():
            sv = s_ref[...]
            runv = runv_ref[...]
            runi = runi_ref[...]
            m = jnp.max(sv, axis=1)[:, None]
            pick = jnp.min(jnp.where(sv == m, colf, _FBIG), axis=1)[:, None]
            s_ref[...] = jnp.where(colf == pick, _NEG, sv)
            pos = jnp.sum(jnp.where(runv >= m, 1.0, 0.0), axis=1)[:, None]
            sh_v = jnp.concatenate([runv[:, :1], runv[:, :-1]], axis=1)
            sh_i = jnp.concatenate([runi[:, :1], runi[:, :-1]], axis=1)
            nv = jnp.where(slotf < pos, runv,
                           jnp.where(slotf == pos, m, sh_v))
            ni = jnp.where(slotf < pos, runi,
                           jnp.where(slotf == pos, pick, sh_i))
            runv_ref[...] = jnp.where(slotf < kf, nv, _NEG)
            runi_ref[...] = jnp.where(slotf < kf, ni, 0.0)

    out_ref[...] = runi_ref[...].astype(jnp.int32)


def _topk(xp, sq, k, d):
    grid = (_NP // _RB, _NP // _CB)
    return pl.pallas_call(
        functools.partial(_topk_body, k=k),
        grid=grid,
        in_specs=[
            pl.BlockSpec((_RB, d), lambda i, j: (i, 0)),
            pl.BlockSpec((_CB, d), lambda i, j: (j, 0)),
            pl.BlockSpec((_RB, 1), lambda i, j: (i, 0)),
            pl.BlockSpec((1, _CB), lambda i, j: (0, j)),
        ],
        out_specs=pl.BlockSpec((_RB, _SLOTS), lambda i, j: (i, 0)),
        out_shape=jax.ShapeDtypeStruct((_NP, _SLOTS), jnp.int32),
        scratch_shapes=[pltpu.VMEM((_RB, _SLOTS), jnp.float32),
                        pltpu.VMEM((_RB, _SLOTS), jnp.float32),
                        pltpu.VMEM((_RB, _CB), jnp.float32)],
        compiler_params=pltpu.CompilerParams(
            dimension_semantics=("arbitrary", "arbitrary")),
    )(xp, xp, sq[:, None], sq[None, :])


def _edge_body(xi_ref, xg_ref, w_ref, b_ref, o_ref, acc_ref, *, k, relu):
    r = pl.program_id(1)
    xi = xi_ref[...]
    xj = xg_ref[0]
    feat = jnp.concatenate([xi, xj - xi], axis=1)
    msg = lax.dot_general(feat, w_ref[...], (((1,), (1,)), ((), ())),
                          preferred_element_type=jnp.float32) + b_ref[...]
    acc = jnp.where(r == 0, msg, jnp.maximum(acc_ref[...], msg))
    acc_ref[...] = acc
    if relu:
        acc = jnp.maximum(acc, jnp.float32(0.0))
    o_ref[...] = acc


def _edge_mlp_max(xi, xg, w, bias, relu):
    k = xg.shape[0]
    d = xi.shape[1]
    ho = w.shape[0]
    return pl.pallas_call(
        functools.partial(_edge_body, k=k, relu=relu),
        grid=(_NP // _RB, k),
        in_specs=[
            pl.BlockSpec((_RB, d), lambda i, r: (i, 0)),
            pl.BlockSpec((1, _RB, d), lambda i, r: (r, i, 0)),
            pl.BlockSpec((ho, 2 * d), lambda i, r: (0, 0)),
            pl.BlockSpec((1, ho), lambda i, r: (0, 0)),
        ],
        out_specs=pl.BlockSpec((_RB, ho), lambda i, r: (i, 0)),
        out_shape=jax.ShapeDtypeStruct((_NP, ho), jnp.float32),
        scratch_shapes=[pltpu.VMEM((_RB, ho), jnp.float32)],
        compiler_params=pltpu.CompilerParams(
            dimension_semantics=("arbitrary", "arbitrary")),
    )(xi, xg, w, bias)


def _sc_gather(src, idx3, k, cp):
    w = src.shape[1]
    info = plsc.get_sparse_core_info()
    nw = info.num_cores * info.num_subcores
    pts = _NP // nw
    steps = pts // cp
    mesh = plsc.VectorSubcoreMesh(core_axis_name="c", subcore_axis_name="s")

    @functools.partial(
        pl.kernel, mesh=mesh,
        out_type=jax.ShapeDtypeStruct((k, _NP, w), jnp.float32),
        scratch_types=[pltpu.VMEM((k * steps, cp), jnp.int32),
                       pltpu.VMEM((cp, w), jnp.float32),
                       pltpu.SemaphoreType.DMA],
    )
    def gk(src_hbm, idx_hbm, out_hbm, idx_v, rows_v, sem):
        wid = lax.axis_index("s") * info.num_cores + lax.axis_index("c")
        base = wid * pts
        pltpu.sync_copy(idx_hbm.at[wid], idx_v)

        for r in range(k):
            def step(si, carry, r=r):
                pltpu.async_copy(
                    src_hbm.at[idx_v.at[r * steps + si]], rows_v, sem).wait()
                pltpu.sync_copy(rows_v,
                                out_hbm.at[r, pl.ds(base + si * cp, cp)])
                return carry

            lax.fori_loop(0, steps, step, 0)

    return gk(src, idx3)


def _arrange_idx(idx, k, cp):
    nw = 32
    pts = _NP // nw
    steps = pts // cp
    idxt = idx[:, :k].T
    return (idxt.reshape(k, nw, steps * cp)
                .transpose(1, 0, 2)
                .reshape(nw, k * steps, cp))


def kernel(x, edge_index, W1, b1, W2, b2):
    del edge_index
    f32 = jnp.float32
    xp = jnp.zeros((_NP, _D), f32).at[:_N].set(x.astype(f32))
    sq1 = jnp.zeros((_NP,), f32).at[:_N].set(jnp.sum(x * x, axis=1))

    idx1 = _topk(xp, sq1, 10, _D)
    idx1v = _arrange_idx(idx1, 10, _CP)
    xg1 = _sc_gather(xp, idx1v, 10, _CP)
    bc1 = b1[None, :]
    h = _edge_mlp_max(xp, xg1, W1, bc1, True)

    sq2 = jnp.zeros((_NP,), f32).at[:_N].set(jnp.sum(h[:_N] * h[:_N], axis=1))
    idx2 = _topk(h, sq2, 8, _H)
    idx2v = _arrange_idx(idx2, 8, _CP)
    hg = _sc_gather(h, idx2v, 8, _CP)
    w2p = jnp.zeros((128, 2 * _H), f32).at[:_C].set(W2)
    bc2 = jnp.zeros((1, 128), f32).at[0, :_C].set(b2)
    out = _edge_mlp_max(h, hg, w2p, bc2, False)
    return out[:_N, :_C]


# speedup vs baseline: 1.2245x; 1.2245x over previous
import functools

import jax
import jax.numpy as jnp
from jax import lax
from jax.experimental import pallas as pl
from jax.experimental.pallas import tpu as pltpu
from jax.experimental.pallas import tpu_sc as plsc

_N = 10000
_D = 128
_H = 256
_C = 2
_NP = 10240
_SLOTS = 16
_RB = 256
_CB = 512
_NEG = -1e30
_FBIG = 2.0**24
_CP = 64


def _topk_body(xr_ref, xc_ref, sqr_ref, sqc_ref, out_ref, runv_ref, runi_ref,
               *, k):
    j = pl.program_id(1)

    @pl.when(j == 0)
    def _():
        runv_ref[...] = jnp.full((_RB, _SLOTS), _NEG, jnp.float32)
        runi_ref[...] = jnp.zeros((_RB, _SLOTS), jnp.float32)

    a = xr_ref[...]
    b = xc_ref[...]
    dot = lax.dot_general(a, b, (((1,), (1,)), ((), ())),
                          preferred_element_type=jnp.float32)
    dist = (sqr_ref[...] + sqc_ref[...]) - 2.0 * dot
    colf = (j * _CB + lax.broadcasted_iota(jnp.int32, (_RB, _CB), 1)
            ).astype(jnp.float32)
    s = -dist + jnp.where(colf >= _N, jnp.float32(_NEG), jnp.float32(0.0))

    runv = runv_ref[...]
    runi = runi_ref[...]
    newv = jnp.full((_RB, _SLOTS), _NEG, jnp.float32)
    newi = jnp.zeros((_RB, _SLOTS), jnp.float32)
    slot = lax.broadcasted_iota(jnp.int32, (_RB, _SLOTS), 1)
    for t in range(k):
        m = jnp.maximum(jnp.max(s, axis=1), jnp.max(runv, axis=1))[:, None]
        pick_s = jnp.min(jnp.where(s == m, colf, _FBIG), axis=1)
        pick_r = jnp.min(jnp.where(runv == m, runi, _FBIG), axis=1)
        pick = jnp.minimum(pick_s, pick_r)[:, None]
        s = jnp.where(colf == pick, _NEG, s)
        runv = jnp.where(runi == pick, _NEG, runv)
        newv = jnp.where(slot == t, m, newv)
        newi = jnp.where(slot == t, pick, newi)
    runv_ref[...] = newv
    runi_ref[...] = newi
    out_ref[...] = newi.astype(jnp.int32)


def _topk(xp, sq, k, d):
    grid = (_NP // _RB, _NP // _CB)
    return pl.pallas_call(
        functools.partial(_topk_body, k=k),
        grid=grid,
        in_specs=[
            pl.BlockSpec((_RB, d), lambda i, j: (i, 0)),
            pl.BlockSpec((_CB, d), lambda i, j: (j, 0)),
            pl.BlockSpec((_RB, 1), lambda i, j: (i, 0)),
            pl.BlockSpec((1, _CB), lambda i, j: (0, j)),
        ],
        out_specs=pl.BlockSpec((_RB, _SLOTS), lambda i, j: (i, 0)),
        out_shape=jax.ShapeDtypeStruct((_NP, _SLOTS), jnp.int32),
        scratch_shapes=[pltpu.VMEM((_RB, _SLOTS), jnp.float32),
                        pltpu.VMEM((_RB, _SLOTS), jnp.float32)],
        compiler_params=pltpu.CompilerParams(
            dimension_semantics=("arbitrary", "arbitrary")),
    )(xp, xp, sq[:, None], sq[None, :])


def _edge_body(xi_ref, xg_ref, w_ref, b_ref, o_ref, acc_ref, *, k, relu):
    r = pl.program_id(1)
    xi = xi_ref[...]
    xj = xg_ref[0]
    feat = jnp.concatenate([xi, xj - xi], axis=1)
    msg = lax.dot_general(feat, w_ref[...], (((1,), (1,)), ((), ())),
                          preferred_element_type=jnp.float32) + b_ref[...]
    acc = jnp.where(r == 0, msg, jnp.maximum(acc_ref[...], msg))
    acc_ref[...] = acc
    if relu:
        acc = jnp.maximum(acc, jnp.float32(0.0))
    o_ref[...] = acc


def _edge_mlp_max(xi, xg, w, bias, relu):
    k = xg.shape[0]
    d = xi.shape[1]
    ho = w.shape[0]
    return pl.pallas_call(
        functools.partial(_edge_body, k=k, relu=relu),
        grid=(_NP // _RB, k),
        in_specs=[
            pl.BlockSpec((_RB, d), lambda i, r: (i, 0)),
            pl.BlockSpec((1, _RB, d), lambda i, r: (r, i, 0)),
            pl.BlockSpec((ho, 2 * d), lambda i, r: (0, 0)),
            pl.BlockSpec((1, ho), lambda i, r: (0, 0)),
        ],
        out_specs=pl.BlockSpec((_RB, ho), lambda i, r: (i, 0)),
        out_shape=jax.ShapeDtypeStruct((_NP, ho), jnp.float32),
        scratch_shapes=[pltpu.VMEM((_RB, ho), jnp.float32)],
        compiler_params=pltpu.CompilerParams(
            dimension_semantics=("arbitrary", "arbitrary")),
    )(xi, xg, w, bias)


def _sc_gather(src, idx3, k, cp):
    w = src.shape[1]
    info = plsc.get_sparse_core_info()
    nw = info.num_cores * info.num_subcores
    pts = _NP // nw
    steps = pts // cp
    mesh = plsc.VectorSubcoreMesh(core_axis_name="c", subcore_axis_name="s")

    @functools.partial(
        pl.kernel, mesh=mesh,
        out_type=jax.ShapeDtypeStruct((k, _NP, w), jnp.float32),
        scratch_types=[pltpu.VMEM((k * steps, cp), jnp.int32),
                       pltpu.VMEM((cp, w), jnp.float32),
                       pltpu.SemaphoreType.DMA],
    )
    def gk(src_hbm, idx_hbm, out_hbm, idx_v, rows_v, sem):
        wid = lax.axis_index("s") * info.num_cores + lax.axis_index("c")
        base = wid * pts
        pltpu.sync_copy(idx_hbm.at[wid], idx_v)

        for r in range(k):
            def step(si, carry, r=r):
                pltpu.async_copy(
                    src_hbm.at[idx_v.at[r * steps + si]], rows_v, sem).wait()
                pltpu.sync_copy(rows_v,
                                out_hbm.at[r, pl.ds(base + si * cp, cp)])
                return carry

            lax.fori_loop(0, steps, step, 0)

    return gk(src, idx3)


def _arrange_idx(idx, k, cp):
    nw = 32
    pts = _NP // nw
    steps = pts // cp
    idxt = idx[:, :k].T
    return (idxt.reshape(k, nw, steps * cp)
                .transpose(1, 0, 2)
                .reshape(nw, k * steps, cp))


def kernel(x, edge_index, W1, b1, W2, b2):
    del edge_index
    f32 = jnp.float32
    xp = jnp.zeros((_NP, _D), f32).at[:_N].set(x.astype(f32))
    sq1 = jnp.zeros((_NP,), f32).at[:_N].set(jnp.sum(x * x, axis=1))

    idx1 = _topk(xp, sq1, 10, _D)
    idx1v = _arrange_idx(idx1, 10, _CP)
    xg1 = _sc_gather(xp, idx1v, 10, _CP)
    bc1 = b1[None, :]
    h = _edge_mlp_max(xp, xg1, W1, bc1, True)

    sq2 = jnp.zeros((_NP,), f32).at[:_N].set(jnp.sum(h[:_N] * h[:_N], axis=1))
    idx2 = _topk(h, sq2, 8, _H)
    idx2v = _arrange_idx(idx2, 8, _CP)
    hg = _sc_gather(h, idx2v, 8, _CP)
    w2p = jnp.zeros((128, 2 * _H), f32).at[:_C].set(W2)
    bc2 = jnp.zeros((1, 128), f32).at[0, :_C].set(b2)
    out = _edge_mlp_max(h, hg, w2p, bc2, False)
    return out[:_N, :_C]


# speedup vs baseline: 1.2383x; 1.0113x over previous
import functools

import jax
import jax.numpy as jnp
from jax import lax
from jax.experimental import pallas as pl
from jax.experimental.pallas import tpu as pltpu
from jax.experimental.pallas import tpu_sc as plsc

_N = 10000
_D = 128
_H = 256
_C = 2
_NP = 10240
_SLOTS = 16
_RB = 256
_CB = 512
_NEG = -1e30
_FBIG = 2.0**24
_CP = 64


def _topk_body(xr_ref, xc_ref, sqr_ref, sqc_ref, out_ref, runv_ref, runi_ref,
               *, k):
    j = pl.program_id(1)

    @pl.when(j == 0)
    def _():
        runv_ref[...] = jnp.full((_RB, _SLOTS), _NEG, jnp.float32)
        runi_ref[...] = jnp.zeros((_RB, _SLOTS), jnp.float32)

    a = xr_ref[...]
    b = xc_ref[...]
    dot = lax.dot_general(a, b, (((1,), (1,)), ((), ())),
                          preferred_element_type=jnp.float32)
    dist = (sqr_ref[...] + sqc_ref[...]) - 2.0 * dot
    colf = (j * _CB + lax.broadcasted_iota(jnp.int32, (_RB, _CB), 1)
            ).astype(jnp.float32)
    s = -dist + jnp.where(colf >= _N, jnp.float32(_NEG), jnp.float32(0.0))

    runv = runv_ref[...]
    runi = runi_ref[...]
    newv = jnp.full((_RB, _SLOTS), _NEG, jnp.float32)
    newi = jnp.zeros((_RB, _SLOTS), jnp.float32)
    slot = lax.broadcasted_iota(jnp.int32, (_RB, _SLOTS), 1)
    for t in range(k):
        m = jnp.maximum(jnp.max(s, axis=1), jnp.max(runv, axis=1))[:, None]
        pick_s = jnp.min(jnp.where(s == m, colf, _FBIG), axis=1)
        pick_r = jnp.min(jnp.where(runv == m, runi, _FBIG), axis=1)
        pick = jnp.minimum(pick_s, pick_r)[:, None]
        s = jnp.where(colf == pick, _NEG, s)
        runv = jnp.where(runi == pick, _NEG, runv)
        newv = jnp.where(slot == t, m, newv)
        newi = jnp.where(slot == t, pick, newi)
    runv_ref[...] = newv
    runi_ref[...] = newi
    out_ref[...] = newi.astype(jnp.int32)


def _topk(xp, sq, k, d):
    grid = (_NP // _RB, _NP // _CB)
    return pl.pallas_call(
        functools.partial(_topk_body, k=k),
        grid=grid,
        in_specs=[
            pl.BlockSpec((_RB, d), lambda i, j: (i, 0)),
            pl.BlockSpec((_CB, d), lambda i, j: (j, 0)),
            pl.BlockSpec((_RB, 1), lambda i, j: (i, 0)),
            pl.BlockSpec((1, _CB), lambda i, j: (0, j)),
        ],
        out_specs=pl.BlockSpec((_RB, _SLOTS), lambda i, j: (i, 0)),
        out_shape=jax.ShapeDtypeStruct((_NP, _SLOTS), jnp.int32),
        scratch_shapes=[pltpu.VMEM((_RB, _SLOTS), jnp.float32),
                        pltpu.VMEM((_RB, _SLOTS), jnp.float32)],
        compiler_params=pltpu.CompilerParams(
            dimension_semantics=("arbitrary", "arbitrary")),
    )(xp, xp, sq[:, None], sq[None, :])


def _edge_body(xi_ref, xg_ref, w_ref, b_ref, o_ref, acc_ref, *, k, relu):
    r = pl.program_id(1)
    xi = xi_ref[...]
    xj = xg_ref[0]
    feat = jnp.concatenate([xi, xj - xi], axis=1)
    msg = lax.dot_general(feat, w_ref[...], (((1,), (1,)), ((), ())),
                          preferred_element_type=jnp.float32) + b_ref[...]
    acc = jnp.where(r == 0, msg, jnp.maximum(acc_ref[...], msg))
    acc_ref[...] = acc
    if relu:
        acc = jnp.maximum(acc, jnp.float32(0.0))
    o_ref[...] = acc


def _edge_mlp_max(xi, xg, w, bias, relu):
    k = xg.shape[0]
    d = xi.shape[1]
    ho = w.shape[0]
    return pl.pallas_call(
        functools.partial(_edge_body, k=k, relu=relu),
        grid=(_NP // _RB, k),
        in_specs=[
            pl.BlockSpec((_RB, d), lambda i, r: (i, 0)),
            pl.BlockSpec((1, _RB, d), lambda i, r: (r, i, 0)),
            pl.BlockSpec((ho, 2 * d), lambda i, r: (0, 0)),
            pl.BlockSpec((1, ho), lambda i, r: (0, 0)),
        ],
        out_specs=pl.BlockSpec((_RB, ho), lambda i, r: (i, 0)),
        out_shape=jax.ShapeDtypeStruct((_NP, ho), jnp.float32),
        scratch_shapes=[pltpu.VMEM((_RB, ho), jnp.float32)],
        compiler_params=pltpu.CompilerParams(
            dimension_semantics=("arbitrary", "arbitrary")),
    )(xi, xg, w, bias)


def _sc_gather(src, idx3, k, cp):
    w = src.shape[1]
    info = plsc.get_sparse_core_info()
    nw = info.num_cores * info.num_subcores
    pts = _NP // nw
    steps = pts // cp
    mesh = plsc.VectorSubcoreMesh(core_axis_name="c", subcore_axis_name="s")

    nbuf = 4
    ntask = k * steps

    @functools.partial(
        pl.kernel, mesh=mesh,
        out_type=jax.ShapeDtypeStruct((k, _NP, w), jnp.float32),
        scratch_types=[pltpu.VMEM((k * steps, cp), jnp.int32),
                       pltpu.VMEM((nbuf, cp, w), jnp.float32)]
                      + [pltpu.SemaphoreType.DMA] * (2 * nbuf),
    )
    def gk(src_hbm, idx_hbm, out_hbm, idx_v, rows_v, *sems):
        sg, sw = sems[:nbuf], sems[nbuf:]
        wid = lax.axis_index("s") * info.num_cores + lax.axis_index("c")
        base = wid * pts
        pltpu.sync_copy(idx_hbm.at[wid], idx_v)

        gathers = [None] * ntask
        writes = [None] * ntask

        def start_gather(t):
            b = t % nbuf
            r, si = divmod(t, steps)
            gathers[t] = pltpu.async_copy(
                src_hbm.at[idx_v.at[r * steps + si]], rows_v.at[b], sg[b])

        start_gather(0)
        for t in range(ntask):
            b = t % nbuf
            if t + 1 < ntask:
                if t + 1 - nbuf >= 0:
                    writes[t + 1 - nbuf].wait()
                start_gather(t + 1)
            gathers[t].wait()
            r, si = divmod(t, steps)
            writes[t] = pltpu.async_copy(
                rows_v.at[b], out_hbm.at[r, pl.ds(base + si * cp, cp)], sw[b])
        for t in range(max(0, ntask - nbuf), ntask):
            writes[t].wait()

    return gk(src, idx3)


def _arrange_idx(idx, k, cp):
    nw = 32
    pts = _NP // nw
    steps = pts // cp
    idxt = idx[:, :k].T
    return (idxt.reshape(k, nw, steps * cp)
                .transpose(1, 0, 2)
                .reshape(nw, k * steps, cp))


def kernel(x, edge_index, W1, b1, W2, b2):
    del edge_index
    f32 = jnp.float32
    xp = jnp.zeros((_NP, _D), f32).at[:_N].set(x.astype(f32))
    sq1 = jnp.zeros((_NP,), f32).at[:_N].set(jnp.sum(x * x, axis=1))

    idx1 = _topk(xp, sq1, 10, _D)
    idx1v = _arrange_idx(idx1, 10, _CP)
    xg1 = _sc_gather(xp, idx1v, 10, _CP)
    bc1 = b1[None, :]
    h = _edge_mlp_max(xp, xg1, W1, bc1, True)

    sq2 = jnp.zeros((_NP,), f32).at[:_N].set(jnp.sum(h[:_N] * h[:_N], axis=1))
    idx2 = _topk(h, sq2, 8, _H)
    idx2v = _arrange_idx(idx2, 8, _CP)
    hg = _sc_gather(h, idx2v, 8, _CP)
    w2p = jnp.zeros((128, 2 * _H), f32).at[:_C].set(W2)
    bc2 = jnp.zeros((1, 128), f32).at[0, :_C].set(b2)
    out = _edge_mlp_max(h, hg, w2p, bc2, False)
    return out[:_N, :_C]


# speedup vs baseline: 1.2395x; 1.0010x over previous
import functools

import jax
import jax.numpy as jnp
from jax import lax
from jax.experimental import pallas as pl
from jax.experimental.pallas import tpu as pltpu
from jax.experimental.pallas import tpu_sc as plsc

_N = 10000
_D = 128
_H = 256
_C = 2
_NP = 10240
_SLOTS = 16
_RB = 256
_CB = 512
_NEG = -1e30
_FBIG = 2.0**24
_CP = 64


def _topk_body(xr_ref, xc_ref, sqr_ref, sqc_ref, out_ref, runv_ref, runi_ref,
               *, k):
    j = pl.program_id(1)

    @pl.when(j == 0)
    def _():
        runv_ref[...] = jnp.full((_RB, _SLOTS), _NEG, jnp.float32)
        runi_ref[...] = jnp.zeros((_RB, _SLOTS), jnp.float32)

    a = xr_ref[...]
    b = xc_ref[...]
    dot = lax.dot_general(a, b, (((1,), (1,)), ((), ())),
                          preferred_element_type=jnp.float32)
    dist = (sqr_ref[...] + sqc_ref[...]) - 2.0 * dot
    colf = (j * _CB + lax.broadcasted_iota(jnp.int32, (_RB, _CB), 1)
            ).astype(jnp.float32)
    s = -dist + jnp.where(colf >= _N, jnp.float32(_NEG), jnp.float32(0.0))

    runv = runv_ref[...]
    runi = runi_ref[...]
    newv = jnp.full((_RB, _SLOTS), _NEG, jnp.float32)
    newi = jnp.zeros((_RB, _SLOTS), jnp.float32)
    slot = lax.broadcasted_iota(jnp.int32, (_RB, _SLOTS), 1)
    for t in range(k):
        m = jnp.maximum(jnp.max(s, axis=1), jnp.max(runv, axis=1))[:, None]
        pick_s = jnp.min(jnp.where(s == m, colf, _FBIG), axis=1)
        pick_r = jnp.min(jnp.where(runv == m, runi, _FBIG), axis=1)
        pick = jnp.minimum(pick_s, pick_r)[:, None]
        s = jnp.where(colf == pick, _NEG, s)
        runv = jnp.where(runi == pick, _NEG, runv)
        newv = jnp.where(slot == t, m, newv)
        newi = jnp.where(slot == t, pick, newi)
    runv_ref[...] = newv
    runi_ref[...] = newi
    out_ref[...] = newi.astype(jnp.int32)


def _topk(xp, sq, k, d):
    grid = (_NP // _RB, _NP // _CB)
    return pl.pallas_call(
        functools.partial(_topk_body, k=k),
        grid=grid,
        in_specs=[
            pl.BlockSpec((_RB, d), lambda i, j: (i, 0)),
            pl.BlockSpec((_CB, d), lambda i, j: (j, 0)),
            pl.BlockSpec((_RB, 1), lambda i, j: (i, 0)),
            pl.BlockSpec((1, _CB), lambda i, j: (0, j)),
        ],
        out_specs=pl.BlockSpec((_RB, _SLOTS), lambda i, j: (i, 0)),
        out_shape=jax.ShapeDtypeStruct((_NP, _SLOTS), jnp.int32),
        scratch_shapes=[pltpu.VMEM((_RB, _SLOTS), jnp.float32),
                        pltpu.VMEM((_RB, _SLOTS), jnp.float32)],
        compiler_params=pltpu.CompilerParams(
            dimension_semantics=("arbitrary", "arbitrary")),
    )(xp, xp, sq[:, None], sq[None, :])


def _edge_body(xi_ref, xg_ref, w_ref, b_ref, o_ref, acc_ref, *, k, relu):
    r = pl.program_id(1)
    xi = xi_ref[...]
    xj = xg_ref[0]
    feat = jnp.concatenate([xi, xj - xi], axis=1)
    msg = lax.dot_general(feat, w_ref[...], (((1,), (1,)), ((), ())),
                          preferred_element_type=jnp.float32) + b_ref[...]
    acc = jnp.where(r == 0, msg, jnp.maximum(acc_ref[...], msg))
    acc_ref[...] = acc
    if relu:
        acc = jnp.maximum(acc, jnp.float32(0.0))
    o_ref[...] = acc


def _edge_mlp_max(xi, xg, w, bias, relu):
    k = xg.shape[0]
    d = xi.shape[1]
    ho = w.shape[0]
    return pl.pallas_call(
        functools.partial(_edge_body, k=k, relu=relu),
        grid=(_NP // _RB, k),
        in_specs=[
            pl.BlockSpec((_RB, d), lambda i, r: (i, 0)),
            pl.BlockSpec((1, _RB, d), lambda i, r: (r, i, 0)),
            pl.BlockSpec((ho, 2 * d), lambda i, r: (0, 0)),
            pl.BlockSpec((1, ho), lambda i, r: (0, 0)),
        ],
        out_specs=pl.BlockSpec((_RB, ho), lambda i, r: (i, 0)),
        out_shape=jax.ShapeDtypeStruct((_NP, ho), jnp.float32),
        scratch_shapes=[pltpu.VMEM((_RB, ho), jnp.float32)],
        compiler_params=pltpu.CompilerParams(
            dimension_semantics=("arbitrary", "arbitrary")),
    )(xi, xg, w, bias)


def _sc_gather(src, idx3, k, cp):
    w = src.shape[1]
    info = plsc.get_sparse_core_info()
    nw = info.num_cores * info.num_subcores
    pts = _NP // nw
    steps = pts // cp
    mesh = plsc.VectorSubcoreMesh(core_axis_name="c", subcore_axis_name="s")

    nbuf = 4
    ntask = k * steps

    @functools.partial(
        pl.kernel, mesh=mesh,
        out_type=jax.ShapeDtypeStruct((k, _NP, w), jnp.float32),
        scratch_types=[pltpu.VMEM((k * steps, cp), jnp.int32),
                       pltpu.VMEM((nbuf, cp, w), jnp.float32)]
                      + [pltpu.SemaphoreType.DMA] * (2 * nbuf),
    )
    def gk(src_hbm, idx_hbm, out_hbm, idx_v, rows_v, *sems):
        sg, sw = sems[:nbuf], sems[nbuf:]
        wid = lax.axis_index("s") * info.num_cores + lax.axis_index("c")
        base = wid * pts
        pltpu.sync_copy(idx_hbm.at[wid], idx_v)

        gathers = [None] * ntask
        writes = [None] * ntask

        def start_gather(t):
            b = t % nbuf
            r, si = divmod(t, steps)
            gathers[t] = pltpu.async_copy(
                src_hbm.at[idx_v.at[r * steps + si]], rows_v.at[b], sg[b])

        start_gather(0)
        for t in range(ntask):
            b = t % nbuf
            if t + 1 < ntask:
                if t + 1 - nbuf >= 0:
                    writes[t + 1 - nbuf].wait()
                start_gather(t + 1)
            gathers[t].wait()
            r, si = divmod(t, steps)
            writes[t] = pltpu.async_copy(
                rows_v.at[b], out_hbm.at[r, pl.ds(base + si * cp, cp)], sw[b])
        for t in range(max(0, ntask - nbuf), ntask):
            writes[t].wait()

    return gk(src, idx3)


def _arrange_idx(idx, k, cp):
    nw = 32
    pts = _NP // nw
    steps = pts // cp
    idxt = idx[:, :k].T
    return (idxt.reshape(k, nw, steps * cp)
                .transpose(1, 0, 2)
                .reshape(nw, k * steps, cp))


def kernel(x, edge_index, W1, b1, W2, b2):
    del edge_index
    f32 = jnp.float32
    xp = jnp.zeros((_NP, _D), f32).at[:_N].set(x.astype(f32))
    sq1 = jnp.sum(xp * xp, axis=1)

    idx1 = _topk(xp, sq1, 10, _D)
    idx1v = _arrange_idx(idx1, 10, _CP)
    xg1 = _sc_gather(xp, idx1v, 10, _CP)
    bc1 = b1[None, :]
    h = _edge_mlp_max(xp, xg1, W1, bc1, True)

    sq2 = jnp.sum(h * h, axis=1)
    idx2 = _topk(h, sq2, 8, _H)
    idx2v = _arrange_idx(idx2, 8, _CP)
    hg = _sc_gather(h, idx2v, 8, _CP)
    w2p = jnp.zeros((128, 2 * _H), f32).at[:_C].set(W2)
    bc2 = jnp.zeros((1, 128), f32).at[0, :_C].set(b2)
    out = _edge_mlp_max(h, hg, w2p, bc2, False)
    return out[:_N, :_C]


# speedup vs baseline: 1.3361x; 1.0779x over previous
import functools

import jax
import jax.numpy as jnp
from jax import lax
from jax.experimental import pallas as pl
from jax.experimental.pallas import tpu as pltpu
from jax.experimental.pallas import tpu_sc as plsc

_N = 10000
_D = 128
_H = 256
_C = 2
_NP = 10240
_SLOTS = 16
_RB = 256
_CB = 512
_NEG = -1e30
_FBIG = 2.0**24
_CP = 32


def _topk_body(xr_ref, xc_ref, sqr_ref, sqc_ref, out_ref, runv_ref, runi_ref,
               *, k):
    j = pl.program_id(1)

    @pl.when(j == 0)
    def _():
        runv_ref[...] = jnp.full((_RB, _SLOTS), _NEG, jnp.float32)
        runi_ref[...] = jnp.zeros((_RB, _SLOTS), jnp.float32)

    a = xr_ref[...]
    b = xc_ref[...]
    dot = lax.dot_general(a, b, (((1,), (1,)), ((), ())),
                          preferred_element_type=jnp.float32)
    dist = (sqr_ref[...] + sqc_ref[...]) - 2.0 * dot
    colf = (j * _CB + lax.broadcasted_iota(jnp.int32, (_RB, _CB), 1)
            ).astype(jnp.float32)
    s = -dist + jnp.where(colf >= _N, jnp.float32(_NEG), jnp.float32(0.0))

    runv = runv_ref[...]
    runi = runi_ref[...]
    newv = jnp.full((_RB, _SLOTS), _NEG, jnp.float32)
    newi = jnp.zeros((_RB, _SLOTS), jnp.float32)
    slot = lax.broadcasted_iota(jnp.int32, (_RB, _SLOTS), 1)
    for t in range(k):
        m = jnp.maximum(jnp.max(s, axis=1), jnp.max(runv, axis=1))[:, None]
        pick_s = jnp.min(jnp.where(s == m, colf, _FBIG), axis=1)
        pick_r = jnp.min(jnp.where(runv == m, runi, _FBIG), axis=1)
        pick = jnp.minimum(pick_s, pick_r)[:, None]
        s = jnp.where(colf == pick, _NEG, s)
        runv = jnp.where(runi == pick, _NEG, runv)
        newv = jnp.where(slot == t, m, newv)
        newi = jnp.where(slot == t, pick, newi)
    runv_ref[...] = newv
    runi_ref[...] = newi
    out_ref[...] = newi.astype(jnp.int32)


def _topk(xr, xc, sqr, sqc, k, d):
    nr = xr.shape[0]
    grid = (nr // _RB, _NP // _CB)
    return pl.pallas_call(
        functools.partial(_topk_body, k=k),
        grid=grid,
        in_specs=[
            pl.BlockSpec((_RB, d), lambda i, j: (i, 0)),
            pl.BlockSpec((_CB, d), lambda i, j: (j, 0)),
            pl.BlockSpec((_RB, 1), lambda i, j: (i, 0)),
            pl.BlockSpec((1, _CB), lambda i, j: (0, j)),
        ],
        out_specs=pl.BlockSpec((_RB, _SLOTS), lambda i, j: (i, 0)),
        out_shape=jax.ShapeDtypeStruct((nr, _SLOTS), jnp.int32),
        scratch_shapes=[pltpu.VMEM((_RB, _SLOTS), jnp.float32),
                        pltpu.VMEM((_RB, _SLOTS), jnp.float32)],
        compiler_params=pltpu.CompilerParams(
            dimension_semantics=("arbitrary", "arbitrary")),
    )(xr, xc, sqr[:, None], sqc[None, :])


def _edge_body(xi_ref, xg_ref, w_ref, b_ref, o_ref, acc_ref, *, k, relu):
    r = pl.program_id(1)
    xi = xi_ref[...]
    xj = xg_ref[0]
    feat = jnp.concatenate([xi, xj - xi], axis=1)
    msg = lax.dot_general(feat, w_ref[...], (((1,), (1,)), ((), ())),
                          preferred_element_type=jnp.float32) + b_ref[...]
    acc = jnp.where(r == 0, msg, jnp.maximum(acc_ref[...], msg))
    acc_ref[...] = acc
    if relu:
        acc = jnp.maximum(acc, jnp.float32(0.0))
    o_ref[...] = acc


def _edge_mlp_max(xi, xg, w, bias, relu):
    k = xg.shape[0]
    nr, d = xi.shape
    ho = w.shape[0]
    return pl.pallas_call(
        functools.partial(_edge_body, k=k, relu=relu),
        grid=(nr // _RB, k),
        in_specs=[
            pl.BlockSpec((_RB, d), lambda i, r: (i, 0)),
            pl.BlockSpec((1, _RB, d), lambda i, r: (r, i, 0)),
            pl.BlockSpec((ho, 2 * d), lambda i, r: (0, 0)),
            pl.BlockSpec((1, ho), lambda i, r: (0, 0)),
        ],
        out_specs=pl.BlockSpec((_RB, ho), lambda i, r: (i, 0)),
        out_shape=jax.ShapeDtypeStruct((nr, ho), jnp.float32),
        scratch_shapes=[pltpu.VMEM((_RB, ho), jnp.float32)],
        compiler_params=pltpu.CompilerParams(
            dimension_semantics=("arbitrary", "arbitrary")),
    )(xi, xg, w, bias)


def _sc_gather(src, idx3, k, cp, nr):
    w = src.shape[1]
    info = plsc.get_sparse_core_info()
    nw = info.num_cores * info.num_subcores
    pts = nr // nw
    steps = pts // cp
    mesh = plsc.VectorSubcoreMesh(core_axis_name="c", subcore_axis_name="s")

    nbuf = 4
    ntask = k * steps

    @functools.partial(
        pl.kernel, mesh=mesh,
        out_type=jax.ShapeDtypeStruct((k, nr, w), jnp.float32),
        scratch_types=[pltpu.VMEM((k * steps, cp), jnp.int32),
                       pltpu.VMEM((nbuf, cp, w), jnp.float32)]
                      + [pltpu.SemaphoreType.DMA] * (2 * nbuf),
    )
    def gk(src_hbm, idx_hbm, out_hbm, idx_v, rows_v, *sems):
        sg, sw = sems[:nbuf], sems[nbuf:]
        wid = lax.axis_index("s") * info.num_cores + lax.axis_index("c")
        base = wid * pts
        pltpu.sync_copy(idx_hbm.at[wid], idx_v)

        gathers = [None] * ntask
        writes = [None] * ntask

        def start_gather(t):
            b = t % nbuf
            r, si = divmod(t, steps)
            gathers[t] = pltpu.async_copy(
                src_hbm.at[idx_v.at[r * steps + si]], rows_v.at[b], sg[b])

        start_gather(0)
        for t in range(ntask):
            b = t % nbuf
            if t + 1 < ntask:
                if t + 1 - nbuf >= 0:
                    writes[t + 1 - nbuf].wait()
                start_gather(t + 1)
            gathers[t].wait()
            r, si = divmod(t, steps)
            writes[t] = pltpu.async_copy(
                rows_v.at[b], out_hbm.at[r, pl.ds(base + si * cp, cp)], sw[b])
        for t in range(max(0, ntask - nbuf), ntask):
            writes[t].wait()

    return gk(src, idx3)


def _arrange_idx(idx, k, cp):
    nw = 32
    nr = idx.shape[0]
    steps = (nr // nw) // cp
    idxt = idx[:, :k].T
    return (idxt.reshape(k, nw, steps * cp)
                .transpose(1, 0, 2)
                .reshape(nw, k * steps, cp))


def kernel(x, edge_index, W1, b1, W2, b2):
    del edge_index
    f32 = jnp.float32
    xp = jnp.zeros((_NP, _D), f32).at[:_N].set(x.astype(f32))
    sq1 = jnp.sum(xp * xp, axis=1)

    half = _NP // 2
    bc1 = b1[None, :]
    w2p = jnp.zeros((128, 2 * _H), f32).at[:_C].set(W2)
    bc2 = jnp.zeros((1, 128), f32).at[0, :_C].set(b2)

    def layer(src, sq, k, w, bias, relu):
        outs = []
        for p in range(2):
            xr = src[p * half:(p + 1) * half]
            idx = _topk(xr, src, sq[p * half:(p + 1) * half], sq, k,
                        src.shape[1])
            idxv = _arrange_idx(idx, k, _CP)
            xg = _sc_gather(src, idxv, k, _CP, half)
            outs.append(_edge_mlp_max(xr, xg, w, bias, relu))
        return jnp.concatenate(outs, axis=0)

    h = layer(xp, sq1, 10, W1, bc1, True)

    sq2 = jnp.sum(h * h, axis=1)
    out = layer(h, sq2, 8, w2p, bc2, False)
    return out[:_N, :_C]


# speedup vs baseline: 1.3364x; 1.0003x over previous
import functools

import jax
import jax.numpy as jnp
from jax import lax
from jax.experimental import pallas as pl
from jax.experimental.pallas import tpu as pltpu
from jax.experimental.pallas import tpu_sc as plsc

_N = 10000
_D = 128
_H = 256
_C = 2
_NP = 10240
_SLOTS = 16
_RB = 256
_CB = 512
_NEG = -1e30
_FBIG = 2.0**24
_CP = 80


def _topk_body(xr_ref, xc_ref, sqr_ref, sqc_ref, out_ref, runv_ref, runi_ref,
               *, k):
    j = pl.program_id(1)

    @pl.when(j == 0)
    def _():
        runv_ref[...] = jnp.full((_RB, _SLOTS), _NEG, jnp.float32)
        runi_ref[...] = jnp.zeros((_RB, _SLOTS), jnp.float32)

    a = xr_ref[...]
    b = xc_ref[...]
    dot = lax.dot_general(a, b, (((1,), (1,)), ((), ())),
                          preferred_element_type=jnp.float32)
    dist = (sqr_ref[...] + sqc_ref[...]) - 2.0 * dot
    colf = (j * _CB + lax.broadcasted_iota(jnp.int32, (_RB, _CB), 1)
            ).astype(jnp.float32)
    s = -dist + jnp.where(colf >= _N, jnp.float32(_NEG), jnp.float32(0.0))

    runv = runv_ref[...]
    runi = runi_ref[...]
    newv = jnp.full((_RB, _SLOTS), _NEG, jnp.float32)
    newi = jnp.zeros((_RB, _SLOTS), jnp.float32)
    slot = lax.broadcasted_iota(jnp.int32, (_RB, _SLOTS), 1)
    for t in range(k):
        m = jnp.maximum(jnp.max(s, axis=1), jnp.max(runv, axis=1))[:, None]
        pick_s = jnp.min(jnp.where(s == m, colf, _FBIG), axis=1)
        pick_r = jnp.min(jnp.where(runv == m, runi, _FBIG), axis=1)
        pick = jnp.minimum(pick_s, pick_r)[:, None]
        s = jnp.where(colf == pick, _NEG, s)
        runv = jnp.where(runi == pick, _NEG, runv)
        newv = jnp.where(slot == t, m, newv)
        newi = jnp.where(slot == t, pick, newi)
    runv_ref[...] = newv
    runi_ref[...] = newi
    out_ref[...] = newi.astype(jnp.int32)


def _topk(xr, xc, sqr, sqc, k, d):
    nr = xr.shape[0]
    grid = (nr // _RB, _NP // _CB)
    return pl.pallas_call(
        functools.partial(_topk_body, k=k),
        grid=grid,
        in_specs=[
            pl.BlockSpec((_RB, d), lambda i, j: (i, 0)),
            pl.BlockSpec((_CB, d), lambda i, j: (j, 0)),
            pl.BlockSpec((_RB, 1), lambda i, j: (i, 0)),
            pl.BlockSpec((1, _CB), lambda i, j: (0, j)),
        ],
        out_specs=pl.BlockSpec((_RB, _SLOTS), lambda i, j: (i, 0)),
        out_shape=jax.ShapeDtypeStruct((nr, _SLOTS), jnp.int32),
        scratch_shapes=[pltpu.VMEM((_RB, _SLOTS), jnp.float32),
                        pltpu.VMEM((_RB, _SLOTS), jnp.float32)],
        compiler_params=pltpu.CompilerParams(
            dimension_semantics=("arbitrary", "arbitrary")),
    )(xr, xc, sqr[:, None], sqc[None, :])


def _edge_body(xi_ref, xg_ref, w_ref, b_ref, o_ref, acc_ref, *, k, relu):
    r = pl.program_id(1)
    xi = xi_ref[...]
    xj = xg_ref[0]
    feat = jnp.concatenate([xi, xj - xi], axis=1)
    msg = lax.dot_general(feat, w_ref[...], (((1,), (1,)), ((), ())),
                          preferred_element_type=jnp.float32) + b_ref[...]
    acc = jnp.where(r == 0, msg, jnp.maximum(acc_ref[...], msg))
    acc_ref[...] = acc
    if relu:
        acc = jnp.maximum(acc, jnp.float32(0.0))
    o_ref[...] = acc


def _edge_mlp_max(xi, xg, w, bias, relu):
    k = xg.shape[0]
    nr, d = xi.shape
    ho = w.shape[0]
    return pl.pallas_call(
        functools.partial(_edge_body, k=k, relu=relu),
        grid=(nr // _RB, k),
        in_specs=[
            pl.BlockSpec((_RB, d), lambda i, r: (i, 0)),
            pl.BlockSpec((1, _RB, d), lambda i, r: (r, i, 0)),
            pl.BlockSpec((ho, 2 * d), lambda i, r: (0, 0)),
            pl.BlockSpec((1, ho), lambda i, r: (0, 0)),
        ],
        out_specs=pl.BlockSpec((_RB, ho), lambda i, r: (i, 0)),
        out_shape=jax.ShapeDtypeStruct((nr, ho), jnp.float32),
        scratch_shapes=[pltpu.VMEM((_RB, ho), jnp.float32)],
        compiler_params=pltpu.CompilerParams(
            dimension_semantics=("arbitrary", "arbitrary")),
    )(xi, xg, w, bias)


def _sc_gather(src, idx3, k, cp, nr):
    w = src.shape[1]
    info = plsc.get_sparse_core_info()
    nw = info.num_cores * info.num_subcores
    pts = nr // nw
    steps = pts // cp
    mesh = plsc.VectorSubcoreMesh(core_axis_name="c", subcore_axis_name="s")

    nbuf = 4
    ntask = k * steps

    @functools.partial(
        pl.kernel, mesh=mesh,
        out_type=jax.ShapeDtypeStruct((k, nr, w), jnp.float32),
        scratch_types=[pltpu.VMEM((k * steps, cp), jnp.int32),
                       pltpu.VMEM((nbuf, cp, w), jnp.float32)]
                      + [pltpu.SemaphoreType.DMA] * (2 * nbuf),
    )
    def gk(src_hbm, idx_hbm, out_hbm, idx_v, rows_v, *sems):
        sg, sw = sems[:nbuf], sems[nbuf:]
        wid = lax.axis_index("s") * info.num_cores + lax.axis_index("c")
        base = wid * pts
        pltpu.sync_copy(idx_hbm.at[wid], idx_v)

        gathers = [None] * ntask
        writes = [None] * ntask

        def start_gather(t):
            b = t % nbuf
            r, si = divmod(t, steps)
            gathers[t] = pltpu.async_copy(
                src_hbm.at[idx_v.at[r * steps + si]], rows_v.at[b], sg[b])

        start_gather(0)
        for t in range(ntask):
            b = t % nbuf
            if t + 1 < ntask:
                if t + 1 - nbuf >= 0:
                    writes[t + 1 - nbuf].wait()
                start_gather(t + 1)
            gathers[t].wait()
            r, si = divmod(t, steps)
            writes[t] = pltpu.async_copy(
                rows_v.at[b], out_hbm.at[r, pl.ds(base + si * cp, cp)], sw[b])
        for t in range(max(0, ntask - nbuf), ntask):
            writes[t].wait()

    return gk(src, idx3)


def _arrange_idx(idx, k, cp):
    nw = 32
    nr = idx.shape[0]
    steps = (nr // nw) // cp
    idxt = idx[:, :k].T
    return (idxt.reshape(k, nw, steps * cp)
                .transpose(1, 0, 2)
                .reshape(nw, k * steps, cp))


def kernel(x, edge_index, W1, b1, W2, b2):
    del edge_index
    f32 = jnp.float32
    xp = jnp.zeros((_NP, _D), f32).at[:_N].set(x.astype(f32))
    sq1 = jnp.sum(xp * xp, axis=1)

    half = _NP // 4
    bc1 = b1[None, :]
    w2p = jnp.zeros((128, 2 * _H), f32).at[:_C].set(W2)
    bc2 = jnp.zeros((1, 128), f32).at[0, :_C].set(b2)

    def layer(src, sq, k, w, bias, relu):
        outs = []
        for p in range(4):
            xr = src[p * half:(p + 1) * half]
            idx = _topk(xr, src, sq[p * half:(p + 1) * half], sq, k,
                        src.shape[1])
            idxv = _arrange_idx(idx, k, _CP)
            xg = _sc_gather(src, idxv, k, _CP, half)
            outs.append(_edge_mlp_max(xr, xg, w, bias, relu))
        return jnp.concatenate(outs, axis=0)

    h = layer(xp, sq1, 10, W1, bc1, True)

    sq2 = jnp.sum(h * h, axis=1)
    out = layer(h, sq2, 8, w2p, bc2, False)
    return out[:_N, :_C]
